# baseline (speedup 1.0000x reference)
import functools

import jax
import jax.numpy as jnp
from jax import lax
from jax.experimental import pallas as pl
from jax.experimental.pallas import tpu as pltpu
from jax.experimental.pallas import tpu_sc as plsc

EMB_DIM = 128
BATCH = 16384
NUM_RANK = 15
NUM_SUIT = 5
NUM_COMB = NUM_RANK * NUM_SUIT

NC = 2
NS = 16
L = 16
NW = NC * NS
BPW = BATCH // NW
CHUNK = 64
NCHUNK = BPW // CHUNK


@functools.partial(
    pl.kernel,
    mesh=plsc.VectorSubcoreMesh(core_axis_name="c", subcore_axis_name="s"),
    out_type=jax.ShapeDtypeStruct((BATCH, EMB_DIM), jnp.float32),
    scratch_types=[
        pltpu.VMEM((BPW,), jnp.int32),
        pltpu.VMEM((BPW,), jnp.int32),
        pltpu.VMEM((NCHUNK, CHUNK), jnp.int32),
        pltpu.VMEM((NCHUNK, CHUNK, EMB_DIM), jnp.float32),
        pltpu.VMEM((1, EMB_DIM), jnp.float32),
        pltpu.VMEM((NUM_SUIT, EMB_DIM), jnp.float32),
        pltpu.VMEM((NUM_SUIT, EMB_DIM), jnp.float32),
        pltpu.VMEM_SHARED((NUM_COMB, EMB_DIM), jnp.float32),
        pltpu.SemaphoreType.DMA((NCHUNK,)),
        pltpu.SemaphoreType.DMA,
        pltpu.SemaphoreType.DMA,
    ],
)
def _sc_lookup(rank_emb_hbm, suit_emb_hbm, rank_hbm, suit_hbm, out_hbm,
               rank_v, suit_v, idx_v, rows_v, rrow_v, stab_v, comb_v,
               table_sp, gsems, isem, osem):
    sid = lax.axis_index("s")
    wid = sid * NC + lax.axis_index("c")
    base = wid * BPW
    ld_r = pltpu.async_copy(rank_hbm.at[pl.ds(base, BPW)], rank_v, isem)
    ld_s = pltpu.async_copy(suit_hbm.at[pl.ds(base, BPW)], suit_v, osem)

    @pl.when(sid < NUM_RANK)
    def _():
        lr = pltpu.async_copy(rank_emb_hbm.at[pl.ds(sid, 1)], rrow_v, gsems.at[0])
        ls = pltpu.async_copy(suit_emb_hbm, stab_v, gsems.at[1])
        lr.wait()
        ls.wait()
        for s in range(NUM_SUIT):
            for c in range(EMB_DIM // L):
                comb_v[s, pl.ds(c * L, L)] = (
                    stab_v[s, pl.ds(c * L, L)] + rrow_v[0, pl.ds(c * L, L)])
        pltpu.sync_copy(comb_v, table_sp.at[pl.ds(sid * NUM_SUIT, NUM_SUIT)])

    ld_r.wait()
    ld_s.wait()
    for i in range(BPW // L):
        j, c = divmod(i, CHUNK // L)
        r = rank_v[pl.ds(i * L, L)]
        s = suit_v[pl.ds(i * L, L)]
        idx_v[j, pl.ds(c * L, L)] = r * NUM_SUIT + s
    plsc.subcore_barrier()
    gathers = [
        pltpu.async_copy(table_sp.at[idx_v.at[j]], rows_v.at[j], gsems.at[j])
        for j in range(NCHUNK)
    ]
    scatters = []
    for j in range(NCHUNK):
        gathers[j].wait()
        scatters.append(pltpu.async_copy(
            rows_v.at[j], out_hbm.at[pl.ds(base + j * CHUNK, CHUNK)], osem))
    for s in scatters:
        s.wait()


def kernel(rank_id, suit_id, rank_emb, suit_emb):
    return _sc_lookup(rank_emb, suit_emb,
                      rank_id.astype(jnp.int32), suit_id.astype(jnp.int32))


# speedup vs baseline: 1.0221x; 1.0221x over previous
import functools

import jax
import jax.numpy as jnp
from jax import lax
from jax.experimental import pallas as pl
from jax.experimental.pallas import tpu as pltpu
from jax.experimental.pallas import tpu_sc as plsc

EMB_DIM = 128
BATCH = 16384
NUM_RANK = 15
NUM_SUIT = 5
NUM_COMB = NUM_RANK * NUM_SUIT

NC = 2
NS = 16
L = 16
NW = NC * NS
BPW = BATCH // NW
CHUNK = 128
NCHUNK = BPW // CHUNK


def _combine_body(rank_ref, suit_ref, out_ref):
    for r in range(NUM_RANK):
        out_ref[pl.ds(r * NUM_SUIT, NUM_SUIT), :] = (
            suit_ref[...] + rank_ref[r, :][None, :])


_combine = pl.pallas_call(
    _combine_body,
    out_shape=jax.ShapeDtypeStruct((NUM_COMB, EMB_DIM), jnp.float32),
)


@functools.partial(
    pl.kernel,
    mesh=plsc.VectorSubcoreMesh(core_axis_name="c", subcore_axis_name="s"),
    out_type=jax.ShapeDtypeStruct((BATCH, EMB_DIM), jnp.float32),
    scratch_types=[
        pltpu.VMEM((BPW,), jnp.int32),
        pltpu.VMEM((BPW,), jnp.int32),
        pltpu.VMEM((NCHUNK, CHUNK), jnp.int32),
        pltpu.VMEM((NCHUNK, CHUNK, EMB_DIM), jnp.float32),
        pltpu.VMEM_SHARED((NUM_COMB, EMB_DIM), jnp.float32),
        pltpu.SemaphoreType.DMA((NCHUNK,)),
        pltpu.SemaphoreType.DMA,
        pltpu.SemaphoreType.DMA,
    ],
)
def _sc_lookup(table_hbm, rank_hbm, suit_hbm, out_hbm,
               rank_v, suit_v, idx_v, rows_v, table_sp, gsems, isem, osem):
    sid = lax.axis_index("s")
    wid = sid * NC + lax.axis_index("c")
    base = wid * BPW
    ld_r = pltpu.async_copy(rank_hbm.at[pl.ds(base, BPW)], rank_v, isem)
    ld_s = pltpu.async_copy(suit_hbm.at[pl.ds(base, BPW)], suit_v, osem)

    @pl.when(sid == 0)
    def _():
        pltpu.sync_copy(table_hbm, table_sp)

    ld_r.wait()
    ld_s.wait()
    for i in range(BPW // L):
        j, c = divmod(i, CHUNK // L)
        r = rank_v[pl.ds(i * L, L)]
        s = suit_v[pl.ds(i * L, L)]
        idx_v[j, pl.ds(c * L, L)] = r * NUM_SUIT + s
    plsc.subcore_barrier()
    gathers = [
        pltpu.async_copy(table_sp.at[idx_v.at[j]], rows_v.at[j], gsems.at[j])
        for j in range(NCHUNK)
    ]
    scatters = []
    for j in range(NCHUNK):
        gathers[j].wait()
        scatters.append(pltpu.async_copy(
            rows_v.at[j], out_hbm.at[pl.ds(base + j * CHUNK, CHUNK)], osem))
    for s in scatters:
        s.wait()


def kernel(rank_id, suit_id, rank_emb, suit_emb):
    comb = _combine(rank_emb, suit_emb)
    return _sc_lookup(comb, rank_id.astype(jnp.int32), suit_id.astype(jnp.int32))


# speedup vs baseline: 1.0282x; 1.0060x over previous
import functools

import jax
import jax.numpy as jnp
from jax import lax
from jax.experimental import pallas as pl
from jax.experimental.pallas import tpu as pltpu
from jax.experimental.pallas import tpu_sc as plsc

EMB_DIM = 128
BATCH = 16384
NUM_RANK = 15
NUM_SUIT = 5
NUM_COMB = NUM_RANK * NUM_SUIT

NC = 2
NS = 16
L = 16
NW = NC * NS
BPW = BATCH // NW
CHUNK = 64
NCHUNK = BPW // CHUNK


def _combine_body(rank_ref, suit_ref, out_ref):
    for r in range(NUM_RANK):
        out_ref[pl.ds(r * NUM_SUIT, NUM_SUIT), :] = (
            suit_ref[...] + rank_ref[r, :][None, :])


_combine = pl.pallas_call(
    _combine_body,
    out_shape=jax.ShapeDtypeStruct((NUM_COMB, EMB_DIM), jnp.float32),
)


@functools.partial(
    pl.kernel,
    mesh=plsc.VectorSubcoreMesh(core_axis_name="c", subcore_axis_name="s"),
    out_type=jax.ShapeDtypeStruct((BATCH, EMB_DIM), jnp.float32),
    scratch_types=[
        pltpu.VMEM((BPW,), jnp.int32),
        pltpu.VMEM((BPW,), jnp.int32),
        pltpu.VMEM((NCHUNK, CHUNK), jnp.int32),
        pltpu.VMEM((NCHUNK, CHUNK, EMB_DIM), jnp.float32),
        pltpu.VMEM_SHARED((NUM_COMB, EMB_DIM), jnp.float32),
        pltpu.SemaphoreType.DMA((NCHUNK,)),
        pltpu.SemaphoreType.DMA,
        pltpu.SemaphoreType.DMA,
    ],
)
def _sc_lookup(table_hbm, rank_hbm, suit_hbm, out_hbm,
               rank_v, suit_v, idx_v, rows_v, table_sp, gsems, isem, osem):
    sid = lax.axis_index("s")
    wid = sid * NC + lax.axis_index("c")
    base = wid * BPW
    ld_r = pltpu.async_copy(rank_hbm.at[pl.ds(base, BPW)], rank_v, isem)
    ld_s = pltpu.async_copy(suit_hbm.at[pl.ds(base, BPW)], suit_v, osem)

    HALF = 40

    @pl.when(sid == 0)
    def _():
        pltpu.sync_copy(table_hbm.at[pl.ds(0, HALF)], table_sp.at[pl.ds(0, HALF)])

    @pl.when(sid == 1)
    def _():
        pltpu.sync_copy(table_hbm.at[pl.ds(HALF, NUM_COMB - HALF)],
                        table_sp.at[pl.ds(HALF, NUM_COMB - HALF)])

    ld_r.wait()
    ld_s.wait()
    CPL = CHUNK // L

    def _idx_body(i, carry):
        j = i // CPL
        c = i - j * CPL
        r = rank_v[pl.ds(i * L, L)]
        s = suit_v[pl.ds(i * L, L)]
        idx_v[j, pl.ds(c * L, L)] = r * NUM_SUIT + s
        return carry

    lax.fori_loop(0, BPW // L, _idx_body, 0, unroll=4)
    plsc.subcore_barrier()
    gathers = [
        pltpu.async_copy(table_sp.at[idx_v.at[j]], rows_v.at[j], gsems.at[j])
        for j in range(NCHUNK)
    ]
    scatters = []
    for j in range(NCHUNK):
        gathers[j].wait()
        scatters.append(pltpu.async_copy(
            rows_v.at[j], out_hbm.at[pl.ds(base + j * CHUNK, CHUNK)], osem))
    for s in scatters:
        s.wait()


def kernel(rank_id, suit_id, rank_emb, suit_emb):
    comb = _combine(rank_emb, suit_emb)
    return _sc_lookup(comb, rank_id.astype(jnp.int32), suit_id.astype(jnp.int32))


# speedup vs baseline: 1.0337x; 1.0054x over previous
import functools

import jax
import jax.numpy as jnp
from jax import lax
from jax.experimental import pallas as pl
from jax.experimental.pallas import tpu as pltpu
from jax.experimental.pallas import tpu_sc as plsc

EMB_DIM = 128
BATCH = 16384
NUM_RANK = 15
NUM_SUIT = 5
NUM_COMB = NUM_RANK * NUM_SUIT

NC = 2
NS = 16
L = 16
NW = NC * NS
BPW = BATCH // NW
CHUNK = 64
NCHUNK = BPW // CHUNK


def _prep_body(rank_ref, suit_ref, rid_ref, sid_ref, comb_ref, idx_ref):
    for r in range(NUM_RANK):
        comb_ref[pl.ds(r * NUM_SUIT, NUM_SUIT), :] = (
            suit_ref[...] + rank_ref[r, :][None, :])
    idx_ref[...] = rid_ref[...] * NUM_SUIT + sid_ref[...]


_prep = pl.pallas_call(
    _prep_body,
    out_shape=(
        jax.ShapeDtypeStruct((NUM_COMB, EMB_DIM), jnp.float32),
        jax.ShapeDtypeStruct((BATCH // EMB_DIM, EMB_DIM), jnp.int32),
    ),
)


@functools.partial(
    pl.kernel,
    mesh=plsc.VectorSubcoreMesh(core_axis_name="c", subcore_axis_name="s"),
    out_type=jax.ShapeDtypeStruct((BATCH, EMB_DIM), jnp.float32),
    scratch_types=[
        pltpu.VMEM((BPW,), jnp.int32),
        pltpu.VMEM((NCHUNK, CHUNK, EMB_DIM), jnp.float32),
        pltpu.VMEM_SHARED((NUM_COMB, EMB_DIM), jnp.float32),
        pltpu.SemaphoreType.DMA((NCHUNK,)),
        pltpu.SemaphoreType.DMA,
        pltpu.SemaphoreType.DMA,
    ],
)
def _sc_lookup(table_hbm, idx_hbm, out_hbm,
               idx_v, rows_v, table_sp, gsems, isem, osem):
    sid = lax.axis_index("s")
    wid = sid * NC + lax.axis_index("c")
    base = wid * BPW
    ld_i = pltpu.async_copy(idx_hbm.at[pl.ds(base, BPW)], idx_v, isem)

    HALF = 40

    @pl.when(sid == 0)
    def _():
        pltpu.sync_copy(table_hbm.at[pl.ds(0, HALF)], table_sp.at[pl.ds(0, HALF)])

    @pl.when(sid == 1)
    def _():
        pltpu.sync_copy(table_hbm.at[pl.ds(HALF, NUM_COMB - HALF)],
                        table_sp.at[pl.ds(HALF, NUM_COMB - HALF)])

    ld_i.wait()
    plsc.subcore_barrier()
    gathers = [
        pltpu.async_copy(table_sp.at[idx_v.at[pl.ds(j * CHUNK, CHUNK)]],
                         rows_v.at[j], gsems.at[j])
        for j in range(NCHUNK)
    ]
    scatters = []
    for j in range(NCHUNK):
        gathers[j].wait()
        scatters.append(pltpu.async_copy(
            rows_v.at[j], out_hbm.at[pl.ds(base + j * CHUNK, CHUNK)], osem))
    for s in scatters:
        s.wait()


def kernel(rank_id, suit_id, rank_emb, suit_emb):
    rid = rank_id.astype(jnp.int32).reshape(BATCH // EMB_DIM, EMB_DIM)
    sid = suit_id.astype(jnp.int32).reshape(BATCH // EMB_DIM, EMB_DIM)
    comb, idx = _prep(rank_emb, suit_emb, rid, sid)
    return _sc_lookup(comb, idx.reshape(BATCH))
